# baseline (speedup 1.0000x reference)
import functools

import jax
import jax.numpy as jnp
from jax import lax
from jax.experimental import pallas as pl
from jax.experimental.pallas import tpu as pltpu
from jax.experimental.pallas import tpu_sc as plsc

N = 10000
E = 320000
SDIM = 128
VDIM = 32

NC, NS = 2, 16
NW = NC * NS
EW = E // NW
CH = 128
NCH = -(-EW // CH)
FULL = EW // CH
TAIL = EW - FULL * CH
STR = N // NS
CW = 8

_MESH = dict(core_axis_name="c", subcore_axis_name="s", num_cores=NC,
             num_subcores=NS)
_PARAMS = pltpu.CompilerParams(use_tc_tiling_on_sc=False)


def _sc_segment_sum_count(e, dst_pad):
    @functools.partial(
        pl.kernel,
        mesh=plsc.VectorSubcoreMesh(**_MESH),
        compiler_params=_PARAMS,
        out_type=(
            jax.ShapeDtypeStruct((NC, N, SDIM), jnp.float32),
            jax.ShapeDtypeStruct((NC, N, CW), jnp.float32),
        ),
        scratch_types=[
            pltpu.VMEM((CH, SDIM), jnp.float32),
            pltpu.VMEM((CH, SDIM), jnp.float32),
            pltpu.VMEM((CH, CW), jnp.float32),
            pltpu.VMEM((NCH, CH), jnp.int32),
            pltpu.VMEM_SHARED((N, SDIM), jnp.float32),
            pltpu.VMEM_SHARED((N, CW), jnp.float32),
            pltpu.SemaphoreType.DMA,
        ],
    )
    def k(e_hbm, dstp_hbm, oute_hbm, outc_hbm, buf0, buf1, ones_v, idx_v,
          acc, accc, sem0):
        c = lax.axis_index("c")
        s = lax.axis_index("s")
        wid = c * NS + s
        ebase = wid * EW
        base = s * STR

        z16 = jnp.zeros((16,), jnp.float32)
        ones16 = jnp.full((16,), 1.0, jnp.float32)

        def zrow(i, carry):
            for g in range(SDIM // 16):
                buf0[i, pl.ds(g * 16, 16)] = z16
            ones_v[i, :] = z16[:CW]
            return carry

        lax.fori_loop(0, CH, zrow, 0)
        for t in range(4):
            pltpu.sync_copy(buf0, acc.at[pl.ds(base + t * CH, CH)])
            pltpu.sync_copy(ones_v, accc.at[pl.ds(base + t * CH, CH)])
        pltpu.sync_copy(buf0.at[pl.ds(0, STR - 4 * CH)],
                        acc.at[pl.ds(base + 4 * CH, STR - 4 * CH)])
        pltpu.sync_copy(ones_v.at[pl.ds(0, STR - 4 * CH)],
                        accc.at[pl.ds(base + 4 * CH, STR - 4 * CH)])

        def orow(i, carry):
            ones_v[i, :] = ones16[:CW]
            return carry

        lax.fori_loop(0, CH, orow, 0)
        pltpu.sync_copy(dstp_hbm.at[wid], idx_v)

        plsc.subcore_barrier()

        pltpu.sync_copy(e_hbm.at[pl.ds(ebase, CH)], buf0)

        def body(i, carry):
            j = 2 * i
            s0 = pltpu.async_copy(buf0, acc.at[idx_v.at[j]], sem0, add=True)
            pltpu.sync_copy(ones_v, accc.at[idx_v.at[j]], add=True)
            pltpu.sync_copy(e_hbm.at[pl.ds(ebase + (j + 1) * CH, CH)], buf1)
            pltpu.sync_copy(buf1, acc.at[idx_v.at[j + 1]], add=True)
            pltpu.sync_copy(ones_v, accc.at[idx_v.at[j + 1]], add=True)
            s0.wait()
            jn = jnp.minimum(j + 2, FULL - 1)
            pltpu.sync_copy(e_hbm.at[pl.ds(ebase + jn * CH, CH)], buf0)
            return carry

        lax.fori_loop(0, FULL // 2, body, 0)

        def ztail(i, carry):
            r = TAIL + i
            for g in range(SDIM // 16):
                buf0[r, pl.ds(g * 16, 16)] = z16
            ones_v[r, :] = z16[:CW]
            return carry

        lax.fori_loop(0, CH - TAIL, ztail, 0)
        pltpu.sync_copy(
            e_hbm.at[pl.ds(ebase + FULL * CH, TAIL)], buf0.at[pl.ds(0, TAIL)]
        )
        pltpu.sync_copy(buf0, acc.at[idx_v.at[FULL]], add=True)
        pltpu.sync_copy(ones_v, accc.at[idx_v.at[FULL]], add=True)

        plsc.subcore_barrier()
        pltpu.sync_copy(acc.at[pl.ds(base, STR)],
                        oute_hbm.at[c, pl.ds(base, STR)])
        pltpu.sync_copy(accc.at[pl.ds(base, STR)],
                        outc_hbm.at[c, pl.ds(base, STR)])

    return k(e, dst_pad)


def _tc_sv_body(s_ref, vt_ref, wst_ref, bs_ref, wv_ref, wpt_ref, bp_ref,
                out_ref):
    s2 = jax.nn.silu(
        jnp.dot(s_ref[...], wst_ref[...], preferred_element_type=jnp.float32)
        + bs_ref[...]
    )
    v3 = jnp.abs(jnp.sum(vt_ref[...] * wv_ref[...], axis=1))
    out_ref[...] = (
        jnp.dot(s2 + v3, wpt_ref[...], preferred_element_type=jnp.float32)
        + bp_ref[...]
    )


def _tc_e_body(parts_ref, cnts_ref, outa_ref, wet_ref, be_ref, wpt_ref,
               out_ref):
    e_sum = parts_ref[0] + parts_ref[1]
    cnt = cnts_ref[0, :, 0:1] + cnts_ref[1, :, 0:1]
    e2 = (0.5 * e_sum) / jnp.maximum(cnt, 1.0)
    e3 = jax.nn.silu(
        jnp.dot(e2, wet_ref[...], preferred_element_type=jnp.float32)
        + be_ref[...]
    )
    out_ref[...] = outa_ref[...] + jnp.dot(
        e3, wpt_ref[...], preferred_element_type=jnp.float32
    )


def _full(shape):
    return pl.BlockSpec(shape, lambda i: tuple(0 for _ in shape))


def _tc_sv(s, vt, wst, bs2, wv3, wpt, bp2, blk):
    return pl.pallas_call(
        _tc_sv_body,
        grid=(N // blk,),
        in_specs=[
            pl.BlockSpec((blk, SDIM), lambda i: (i, 0)),
            pl.BlockSpec((blk, VDIM, SDIM), lambda i: (i, 0, 0)),
            _full((SDIM, SDIM)),
            _full((1, SDIM)),
            _full((1, VDIM, 1)),
            _full((SDIM, 1)),
            _full((1, 1)),
        ],
        out_specs=pl.BlockSpec((blk, 1), lambda i: (i, 0)),
        out_shape=jax.ShapeDtypeStruct((N, 1), jnp.float32),
    )(s, vt, wst, bs2, wv3, wpt, bp2)


def _tc_e(parts, cnts, outa, wet, be2, wpt, blk):
    return pl.pallas_call(
        _tc_e_body,
        grid=(N // blk,),
        in_specs=[
            pl.BlockSpec((NC, blk, SDIM), lambda i: (0, i, 0)),
            pl.BlockSpec((NC, blk, CW), lambda i: (0, i, 0)),
            pl.BlockSpec((blk, 1), lambda i: (i, 0)),
            _full((SDIM, SDIM)),
            _full((1, SDIM)),
            _full((SDIM, 1)),
        ],
        out_specs=pl.BlockSpec((blk, 1), lambda i: (i, 0)),
        out_shape=jax.ShapeDtypeStruct((N, 1), jnp.float32),
    )(parts, cnts, outa, wet, be2, wpt)


def kernel(s, v, p, e, batch, edge_index_global, W_s, b_s, W_e, b_e, W_v, W_p, b_p):
    del p, batch
    dst = edge_index_global[1].astype(jnp.int32)

    dst_r = dst.reshape(NW, EW)
    pad = NCH * CH - EW
    padidx = jnp.zeros((NW, 1), jnp.int32) + jnp.arange(pad, dtype=jnp.int32)
    dst_pad = jnp.concatenate([dst_r, padidx], axis=1).reshape(NW, NCH, CH)

    parts, partc = _sc_segment_sum_count(e, dst_pad)

    vt = jnp.transpose(v, (0, 2, 1))
    wv3 = W_v.reshape(1, VDIM, 1).astype(jnp.float32)

    outa = _tc_sv(
        s, vt, W_s.T, b_s.reshape(1, SDIM), wv3, W_p.T, b_p.reshape(1, 1),
        blk=1000,
    )
    return _tc_e(
        parts, partc, outa, W_e.T, b_e.reshape(1, SDIM), W_p.T, blk=1000
    )


# speedup vs baseline: 1.1190x; 1.1190x over previous
import functools

import jax
import jax.numpy as jnp
from jax import lax
from jax.experimental import pallas as pl
from jax.experimental.pallas import tpu as pltpu
from jax.experimental.pallas import tpu_sc as plsc

N = 10000
E = 320000
SDIM = 128
VDIM = 32

NC, NS = 2, 16
NW = NC * NS
EW = E // NW
CH = 128
NCH = -(-EW // CH)
FULL = EW // CH
TAIL = EW - FULL * CH
R = NW * ((NCH * CH) // NW)
STR = R // NS
CW = 16

_MESH = dict(core_axis_name="c", subcore_axis_name="s", num_cores=NC,
             num_subcores=NS)
_PARAMS = pltpu.CompilerParams(use_tc_tiling_on_sc=False)


def _sc_segment_sum(e, dst_pad):
    @functools.partial(
        pl.kernel,
        mesh=plsc.VectorSubcoreMesh(**_MESH),
        compiler_params=_PARAMS,
        out_type=jax.ShapeDtypeStruct((NC, R, SDIM), jnp.float32),
        scratch_types=[
            pltpu.VMEM((CH, SDIM), jnp.float32),
            pltpu.VMEM((CH, SDIM), jnp.float32),
            pltpu.VMEM((NCH, CH), jnp.int32),
            pltpu.VMEM_SHARED((R, SDIM), jnp.float32),
            pltpu.SemaphoreType.DMA,
            pltpu.SemaphoreType.DMA,
        ],
    )
    def k(e_hbm, dstp_hbm, out_hbm, buf0, buf1, idx_v, acc, sem0, sem1):
        c = lax.axis_index("c")
        s = lax.axis_index("s")
        wid = c * NS + s
        ebase = wid * EW

        z16 = jnp.zeros((16,), jnp.float32)

        def zrow(i, carry):
            for g in range(SDIM // 16):
                buf0[i, pl.ds(g * 16, 16)] = z16
            return carry

        lax.fori_loop(0, CH, zrow, 0)
        base = s * STR
        for t in range(4):
            pltpu.sync_copy(buf0, acc.at[pl.ds(base + t * CH, CH)])
        pltpu.sync_copy(buf0.at[pl.ds(0, STR - 4 * CH)],
                        acc.at[pl.ds(base + 4 * CH, STR - 4 * CH)])
        pltpu.sync_copy(dstp_hbm.at[wid], idx_v)

        plsc.subcore_barrier()

        pltpu.sync_copy(e_hbm.at[pl.ds(ebase, CH)], buf0)

        def body(i, carry):
            j = 2 * i
            s0 = pltpu.async_copy(buf0, acc.at[idx_v.at[j]], sem0, add=True)
            pltpu.sync_copy(e_hbm.at[pl.ds(ebase + (j + 1) * CH, CH)], buf1)
            s1 = pltpu.async_copy(buf1, acc.at[idx_v.at[j + 1]], sem1,
                                  add=True)
            s0.wait()
            jn = jnp.minimum(j + 2, FULL - 1)
            pltpu.sync_copy(e_hbm.at[pl.ds(ebase + jn * CH, CH)], buf0)
            s1.wait()
            return carry

        lax.fori_loop(0, FULL // 2, body, 0)

        pltpu.sync_copy(
            e_hbm.at[pl.ds(ebase + FULL * CH, TAIL)], buf0.at[pl.ds(0, TAIL)]
        )
        pltpu.sync_copy(buf0, acc.at[idx_v.at[FULL]], add=True)

        plsc.subcore_barrier()
        pltpu.sync_copy(
            acc.at[pl.ds(s * STR, STR)], out_hbm.at[c, pl.ds(s * STR, STR)]
        )

    return k(e, dst_pad)


def _sc_segment_count(dst_pad):
    @functools.partial(
        pl.kernel,
        mesh=plsc.VectorSubcoreMesh(**_MESH),
        compiler_params=_PARAMS,
        out_type=jax.ShapeDtypeStruct((NC, R, CW), jnp.float32),
        scratch_types=[
            pltpu.VMEM((CH, CW), jnp.float32),
            pltpu.VMEM((NCH, CH), jnp.int32),
            pltpu.VMEM_SHARED((R, CW), jnp.float32),
        ],
    )
    def k(dstp_hbm, out_hbm, ones_v, idx_v, acc):
        c = lax.axis_index("c")
        s = lax.axis_index("s")
        wid = c * NS + s

        z16 = jnp.zeros((16,), jnp.float32)

        def zrow(i, carry):
            ones_v[i, :] = z16
            return carry

        lax.fori_loop(0, CH, zrow, 0)
        base = s * STR
        for t in range(4):
            pltpu.sync_copy(ones_v, acc.at[pl.ds(base + t * CH, CH)])
        pltpu.sync_copy(ones_v.at[pl.ds(0, STR - 4 * CH)],
                        acc.at[pl.ds(base + 4 * CH, STR - 4 * CH)])
        pltpu.sync_copy(dstp_hbm.at[wid], idx_v)

        ones16 = jnp.full((16,), 1.0, jnp.float32)

        def initb(i, carry):
            ones_v[i, :] = ones16
            return carry

        lax.fori_loop(0, CH, initb, 0)
        plsc.subcore_barrier()

        def body(j, carry):
            pltpu.sync_copy(ones_v, acc.at[idx_v.at[j]], add=True)
            return carry

        lax.fori_loop(0, NCH, body, 0)

        plsc.subcore_barrier()
        pltpu.sync_copy(
            acc.at[pl.ds(s * STR, STR)], out_hbm.at[c, pl.ds(s * STR, STR)]
        )

    return k(dst_pad)


def _tc_sv_body(s_ref, vt_ref, wst_ref, bs_ref, wv_ref, wpt_ref, bp_ref,
                out_ref):
    s2 = jax.nn.silu(
        jnp.dot(s_ref[...], wst_ref[...], preferred_element_type=jnp.float32)
        + bs_ref[...]
    )
    v3 = jnp.abs(jnp.sum(vt_ref[...] * wv_ref[...], axis=1))
    out_ref[...] = (
        jnp.dot(s2 + v3, wpt_ref[...], preferred_element_type=jnp.float32)
        + bp_ref[...]
    )


def _tc_e_body(parts_ref, cnts_ref, outa_ref, wet_ref, be_ref, wpt_ref,
               out_ref):
    e_sum = parts_ref[0] + parts_ref[1]
    cnt = cnts_ref[0, :, 0:1] + cnts_ref[1, :, 0:1]
    e2 = (0.5 * e_sum) / jnp.maximum(cnt, 1.0)
    e3 = jax.nn.silu(
        jnp.dot(e2, wet_ref[...], preferred_element_type=jnp.float32)
        + be_ref[...]
    )
    out_ref[...] = outa_ref[...] + jnp.dot(
        e3, wpt_ref[...], preferred_element_type=jnp.float32
    )


def _full(shape):
    return pl.BlockSpec(shape, lambda i: tuple(0 for _ in shape))


def _tc_sv(s, vt, wst, bs2, wv3, wpt, bp2, blk):
    return pl.pallas_call(
        _tc_sv_body,
        grid=(N // blk,),
        in_specs=[
            pl.BlockSpec((blk, SDIM), lambda i: (i, 0)),
            pl.BlockSpec((blk, VDIM, SDIM), lambda i: (i, 0, 0)),
            _full((SDIM, SDIM)),
            _full((1, SDIM)),
            _full((1, VDIM, 1)),
            _full((SDIM, 1)),
            _full((1, 1)),
        ],
        out_specs=pl.BlockSpec((blk, 1), lambda i: (i, 0)),
        out_shape=jax.ShapeDtypeStruct((N, 1), jnp.float32),
    )(s, vt, wst, bs2, wv3, wpt, bp2)


def _tc_e(parts, cnts, outa, wet, be2, wpt, blk):
    return pl.pallas_call(
        _tc_e_body,
        grid=(N // blk,),
        in_specs=[
            pl.BlockSpec((NC, blk, SDIM), lambda i: (0, i, 0)),
            pl.BlockSpec((NC, blk, CW), lambda i: (0, i, 0)),
            pl.BlockSpec((blk, 1), lambda i: (i, 0)),
            _full((SDIM, SDIM)),
            _full((1, SDIM)),
            _full((SDIM, 1)),
        ],
        out_specs=pl.BlockSpec((blk, 1), lambda i: (i, 0)),
        out_shape=jax.ShapeDtypeStruct((N, 1), jnp.float32),
    )(parts, cnts, outa, wet, be2, wpt)


def kernel(s, v, p, e, batch, edge_index_global, W_s, b_s, W_e, b_e, W_v, W_p, b_p):
    del p, batch
    dst = edge_index_global[1].astype(jnp.int32)

    dst_r = dst.reshape(NW, EW)
    pad = NCH * CH - EW
    trash = (N + jnp.arange(NW, dtype=jnp.int32))[:, None] + jnp.zeros(
        (1, pad), jnp.int32
    )
    dst_pad = jnp.concatenate([dst_r, trash], axis=1).reshape(NW, NCH, CH)

    parts = _sc_segment_sum(e, dst_pad)
    partc = _sc_segment_count(dst_pad)

    vt = jnp.transpose(v, (0, 2, 1))
    wv3 = W_v.reshape(1, VDIM, 1).astype(jnp.float32)

    outa = _tc_sv(
        s, vt, W_s.T, b_s.reshape(1, SDIM), wv3, W_p.T, b_p.reshape(1, 1),
        blk=1000,
    )
    return _tc_e(
        parts, partc, outa, W_e.T, b_e.reshape(1, SDIM), W_p.T, blk=1000
    )
